# baseline (speedup 1.0000x reference)
import functools

import jax
import jax.numpy as jnp
import numpy as np
from jax import lax
from jax.experimental import pallas as pl
from jax.experimental.pallas import tpu as pltpu
from jax.experimental.pallas import tpu_sc as plsc

_OFFS = tuple((dr, dc) for dr in (-1, 0, 1) for dc in (-1, 0, 1))


def _roll_lanes(a, shift):
    s = shift % a.shape[-1]
    if s == 0:
        return a
    return jnp.concatenate([a[:, s:], a[:, :s]], axis=1)


def _attn_body(x_ref, wq_ref, bq_ref, wk_ref, bk_ref, s_ref,
               attn_ref, vals_ref, *, width, height):
    n_pos = width * height
    xf = x_ref[0]
    q = jnp.dot(wq_ref[...], xf, preferred_element_type=jnp.float32) + bq_ref[...]
    k = jnp.dot(wk_ref[...], xf, preferred_element_type=jnp.float32) + bk_ref[...]

    n_iota = jax.lax.broadcasted_iota(jnp.int32, (1, n_pos), 1)
    r = n_iota // height
    c = n_iota % height

    energies = []
    for dr, dc in _OFFS:
        off = dr * height + dc
        kr = _roll_lanes(k, off)
        e = jnp.sum(q * kr, axis=0, keepdims=True)
        valid = ((r + dr >= 0) & (r + dr < width)
                 & (c + dc >= 0) & (c + dc < height))
        energies.append(jnp.where(valid, e, -1e30))
    energy = jnp.concatenate(energies, axis=0)
    emax = jnp.max(energy, axis=0, keepdims=True)
    p = jnp.exp(energy - emax)
    attn = p / jnp.sum(p, axis=0, keepdims=True)
    attn16 = jnp.concatenate(
        [attn, jnp.zeros((16 - attn.shape[0], n_pos), jnp.float32)], axis=0)
    attn_ref[0] = attn16

    attn_t = lax.dot_general(
        attn16, s_ref[...], (((0,), (0,)), ((), ())),
        preferred_element_type=jnp.float32)
    nrow = jax.lax.broadcasted_iota(jnp.int32, (n_pos, 128), 0)
    jcol = jax.lax.broadcasted_iota(jnp.int32, (n_pos, 128), 1)
    wb = jnp.minimum(jnp.maximum(((nrow - 33) // 16) * 16, 0), n_pos - 128)
    c0 = jcol + wb - nrow
    vals = jnp.zeros((n_pos, 128), jnp.float32)
    for i, (dr, dc) in enumerate(_OFFS):
        off = dr * height + dc
        vals = jnp.where(c0 == off, attn_t[:, i:i + 1], vals)
    vals_ref[0] = vals


def _out_body(x_ref, attn_ref, wv_ref, bv_ref, g_ref, out_ref,
              *, width, height):
    xf = x_ref[0]
    v = jnp.dot(wv_ref[...], xf, preferred_element_type=jnp.float32) + bv_ref[...]
    attn = attn_ref[0]
    acc = jnp.zeros_like(v)
    for i, (dr, dc) in enumerate(_OFFS):
        off = dr * height + dc
        acc = acc + attn[i:i + 1, :] * _roll_lanes(v, off)
    res = g_ref[0, 0] * acc + xf
    out_ref[0] = res.reshape(res.shape[0], width, height)


def _attn_call(xf, Wq, bq, Wk, bk, width, height):
    B, C, N = xf.shape
    d = Wq.shape[0]
    body = functools.partial(_attn_body, width=width, height=height)
    return pl.pallas_call(
        body,
        grid=(B,),
        in_specs=[
            pl.BlockSpec((1, C, N), lambda b: (b, 0, 0)),
            pl.BlockSpec((d, C), lambda b: (0, 0)),
            pl.BlockSpec((d, 1), lambda b: (0, 0)),
            pl.BlockSpec((d, C), lambda b: (0, 0)),
            pl.BlockSpec((d, 1), lambda b: (0, 0)),
            pl.BlockSpec((16, 16), lambda b: (0, 0)),
        ],
        out_specs=[
            pl.BlockSpec((1, 16, N), lambda b: (b, 0, 0)),
            pl.BlockSpec((1, N, 128), lambda b: (b, 0, 0)),
        ],
        out_shape=[
            jax.ShapeDtypeStruct((B, 16, N), jnp.float32),
            jax.ShapeDtypeStruct((B, N, 128), jnp.float32),
        ],
    )(xf, Wq, bq.reshape(d, 1), Wk, bk.reshape(d, 1),
      jnp.eye(16, dtype=jnp.float32))


def _out_call(xf, attn16, Wv, bv, gamma, width, height):
    B, C, N = xf.shape
    body = functools.partial(_out_body, width=width, height=height)
    return pl.pallas_call(
        body,
        grid=(B,),
        in_specs=[
            pl.BlockSpec((1, C, N), lambda b: (b, 0, 0)),
            pl.BlockSpec((1, 16, N), lambda b: (b, 0, 0)),
            pl.BlockSpec((C, C), lambda b: (0, 0)),
            pl.BlockSpec((C, 1), lambda b: (0, 0)),
            pl.BlockSpec((1, 1), lambda b: (0, 0)),
        ],
        out_specs=pl.BlockSpec((1, C, width, height), lambda b: (b, 0, 0, 0)),
        out_shape=jax.ShapeDtypeStruct((B, C, width, height), jnp.float32),
    )(xf, attn16, Wv, bv.reshape(C, 1), gamma.reshape(1, 1))


_RB = 32
_NCHUNK = 8


def _make_sc_builder(B, N):
    info = plsc.get_sparse_core_info()
    NC, NS = info.num_cores, info.num_subcores
    NW = NC * NS
    rows_w = (B * N) // NW
    mesh = plsc.VectorSubcoreMesh(core_axis_name="c", subcore_axis_name="s")

    @functools.partial(
        pl.kernel, mesh=mesh,
        out_type=jax.ShapeDtypeStruct((B * N, N), jnp.float32),
        scratch_types=[
            pltpu.VMEM((rows_w, 128), jnp.float32),
            pltpu.VMEM((2 * _RB, N), jnp.float32),
            pltpu.SemaphoreType.DMA,
            pltpu.SemaphoreType.DMA,
        ],
    )
    def build(vals_hbm, zeros_hbm, att_hbm, vals_v, rows_v, sem0, sem1):
        wid = lax.axis_index("s") * NC + lax.axis_index("c")
        base = wid * rows_w
        m0 = base % N

        pltpu.sync_copy(vals_hbm.at[pl.ds(base, rows_w), :], vals_v)
        pltpu.sync_copy(zeros_hbm, rows_v.at[pl.ds(0, _RB), :])
        pltpu.sync_copy(zeros_hbm, rows_v.at[pl.ds(_RB, _RB), :])

        zeros16 = jnp.zeros((16,), jnp.float32)
        sems = (sem0, sem1)

        def wclamp(m):
            wb = jnp.minimum(jnp.maximum(((m - 33) // 16) * 16, 0), N - 128)
            return pl.multiple_of(wb, 16)

        def build_chunk(c, bufrow0):
            def body(row, carry):
                rr = c * _RB + row
                wb = wclamp(m0 + rr)
                for t in range(8):
                    g = vals_v[rr, pl.ds(16 * t, 16)]
                    rows_v[bufrow0 + row, pl.ds(wb + 16 * t, 16)] = g
                return carry
            lax.fori_loop(0, _RB, body, 0)

        def clear_chunk(c, bufrow0):
            def body(row, carry):
                rr = c * _RB + row
                wb = wclamp(m0 + rr)
                for t in range(8):
                    rows_v[bufrow0 + row, pl.ds(wb + 16 * t, 16)] = zeros16
                return carry
            lax.fori_loop(0, _RB, body, 0)

        handles = {}
        for c in range(_NCHUNK):
            bufrow0 = (c % 2) * _RB
            if c >= 2:
                handles[c - 2].wait()
                clear_chunk(c - 2, bufrow0)
            build_chunk(c, bufrow0)
            handles[c] = pltpu.async_copy(
                rows_v.at[pl.ds(bufrow0, _RB), :],
                att_hbm.at[pl.ds(base + c * _RB, _RB), :],
                sems[c % 2])
        handles[_NCHUNK - 2].wait()
        handles[_NCHUNK - 1].wait()

    return build


def kernel(x, Wq, bq, Wk, bk, Wv, bv, gamma):
    B, C, width, height = x.shape
    N = width * height
    xf = x.reshape(B, C, N)
    attn16, vals = _attn_call(xf, Wq, bq, Wk, bk, width, height)
    zeros = jnp.zeros((_RB, N), jnp.float32)
    att = _make_sc_builder(B, N)(vals.reshape(B * N, 128), zeros)
    out3 = _out_call(xf, attn16, Wv, bv, gamma, width, height)
    return out3, att.reshape(B, N, N)


# speedup vs baseline: 1.4265x; 1.4265x over previous
import functools

import jax
import jax.numpy as jnp
import numpy as np
from jax import lax
from jax.experimental import pallas as pl
from jax.experimental.pallas import tpu as pltpu
from jax.experimental.pallas import tpu_sc as plsc

_OFFS = tuple((dr, dc) for dr in (-1, 0, 1) for dc in (-1, 0, 1))


def _roll_lanes(a, shift):
    s = shift % a.shape[-1]
    if s == 0:
        return a
    return jnp.concatenate([a[:, s:], a[:, :s]], axis=1)


def _attn_body(x_ref, wq_ref, bq_ref, wk_ref, bk_ref, s_ref,
               attn_ref, vals_ref, *, width, height):
    n_pos = width * height
    xf = x_ref[0]
    q = jnp.dot(wq_ref[...], xf, preferred_element_type=jnp.float32) + bq_ref[...]
    k = jnp.dot(wk_ref[...], xf, preferred_element_type=jnp.float32) + bk_ref[...]

    n_iota = jax.lax.broadcasted_iota(jnp.int32, (1, n_pos), 1)
    r = n_iota // height
    c = n_iota % height

    energies = []
    for dr, dc in _OFFS:
        off = dr * height + dc
        kr = _roll_lanes(k, off)
        e = jnp.sum(q * kr, axis=0, keepdims=True)
        valid = ((r + dr >= 0) & (r + dr < width)
                 & (c + dc >= 0) & (c + dc < height))
        energies.append(jnp.where(valid, e, -1e30))
    energy = jnp.concatenate(energies, axis=0)
    emax = jnp.max(energy, axis=0, keepdims=True)
    p = jnp.exp(energy - emax)
    attn = p / jnp.sum(p, axis=0, keepdims=True)
    attn16 = jnp.concatenate(
        [attn, jnp.zeros((16 - attn.shape[0], n_pos), jnp.float32)], axis=0)
    attn_ref[0] = attn16

    attn_t = lax.dot_general(
        attn16, s_ref[...], (((0,), (0,)), ((), ())),
        preferred_element_type=jnp.float32)
    nrow = jax.lax.broadcasted_iota(jnp.int32, (n_pos, 128), 0)
    jcol = jax.lax.broadcasted_iota(jnp.int32, (n_pos, 128), 1)
    wb = jnp.minimum(jnp.maximum(((nrow - 33) // 16) * 16, 0), n_pos - 128)
    c0 = jcol + wb - nrow
    vals = jnp.zeros((n_pos, 128), jnp.float32)
    for i, (dr, dc) in enumerate(_OFFS):
        off = dr * height + dc
        vals = jnp.where(c0 == off, attn_t[:, i:i + 1], vals)
    vals_ref[0] = vals


def _out_body(x_ref, attn_ref, wv_ref, bv_ref, g_ref, out_ref,
              *, width, height):
    xf = x_ref[0]
    v = jnp.dot(wv_ref[...], xf, preferred_element_type=jnp.float32) + bv_ref[...]
    attn = attn_ref[0]
    acc = jnp.zeros_like(v)
    for i, (dr, dc) in enumerate(_OFFS):
        off = dr * height + dc
        acc = acc + attn[i:i + 1, :] * _roll_lanes(v, off)
    out_ref[0] = g_ref[0, 0] * acc + xf


def _attn_call(xf, Wq, bq, Wk, bk, width, height):
    B, C, N = xf.shape
    d = Wq.shape[0]
    body = functools.partial(_attn_body, width=width, height=height)
    return pl.pallas_call(
        body,
        grid=(B,),
        in_specs=[
            pl.BlockSpec((1, C, N), lambda b: (b, 0, 0)),
            pl.BlockSpec((d, C), lambda b: (0, 0)),
            pl.BlockSpec((d, 1), lambda b: (0, 0)),
            pl.BlockSpec((d, C), lambda b: (0, 0)),
            pl.BlockSpec((d, 1), lambda b: (0, 0)),
            pl.BlockSpec((16, 16), lambda b: (0, 0)),
        ],
        out_specs=[
            pl.BlockSpec((1, 16, N), lambda b: (b, 0, 0)),
            pl.BlockSpec((1, N, 128), lambda b: (b, 0, 0)),
        ],
        out_shape=[
            jax.ShapeDtypeStruct((B, 16, N), jnp.float32),
            jax.ShapeDtypeStruct((B, N, 128), jnp.float32),
        ],
    )(xf, Wq, bq.reshape(d, 1), Wk, bk.reshape(d, 1),
      jnp.eye(16, dtype=jnp.float32))


def _out_call(xf, attn16, Wv, bv, gamma, width, height):
    B, C, N = xf.shape
    body = functools.partial(_out_body, width=width, height=height)
    return pl.pallas_call(
        body,
        grid=(B,),
        in_specs=[
            pl.BlockSpec((1, C, N), lambda b: (b, 0, 0)),
            pl.BlockSpec((1, 16, N), lambda b: (b, 0, 0)),
            pl.BlockSpec((C, C), lambda b: (0, 0)),
            pl.BlockSpec((C, 1), lambda b: (0, 0)),
            pl.BlockSpec((1, 1), lambda b: (0, 0)),
        ],
        out_specs=pl.BlockSpec((1, C, N), lambda b: (b, 0, 0)),
        out_shape=jax.ShapeDtypeStruct((B, C, N), jnp.float32),
    )(xf, attn16, Wv, bv.reshape(C, 1), gamma.reshape(1, 1))


_RB = 32
_NCHUNK = 8


def _make_sc_builder(B, N):
    info = plsc.get_sparse_core_info()
    NC, NS = info.num_cores, info.num_subcores
    NW = NC * NS
    rows_w = (B * N) // NW
    mesh = plsc.VectorSubcoreMesh(core_axis_name="c", subcore_axis_name="s")

    @functools.partial(
        pl.kernel, mesh=mesh,
        out_type=jax.ShapeDtypeStruct((B * N, N), jnp.float32),
        scratch_types=[
            pltpu.VMEM((rows_w, 128), jnp.float32),
            pltpu.VMEM((2 * _RB, N), jnp.float32),
            pltpu.SemaphoreType.DMA,
            pltpu.SemaphoreType.DMA,
        ],
    )
    def build(vals_hbm, zeros_hbm, att_hbm, vals_v, rows_v, sem0, sem1):
        wid = lax.axis_index("s") * NC + lax.axis_index("c")
        base = wid * rows_w
        m0 = base % N

        pltpu.sync_copy(vals_hbm.at[pl.ds(base, rows_w), :], vals_v)
        pltpu.sync_copy(zeros_hbm, rows_v.at[pl.ds(0, _RB), :])
        pltpu.sync_copy(zeros_hbm, rows_v.at[pl.ds(_RB, _RB), :])

        zeros16 = jnp.zeros((16,), jnp.float32)
        sems = (sem0, sem1)

        def wclamp(m):
            wb = jnp.minimum(jnp.maximum(((m - 33) // 16) * 16, 0), N - 128)
            return pl.multiple_of(wb, 16)

        def build_chunk(c, bufrow0):
            def body(row, carry):
                rr = c * _RB + row
                wb = wclamp(m0 + rr)
                for t in range(8):
                    g = vals_v[rr, pl.ds(16 * t, 16)]
                    rows_v[bufrow0 + row, pl.ds(wb + 16 * t, 16)] = g
                return carry
            lax.fori_loop(0, _RB, body, 0)

        def clear_chunk(c, bufrow0):
            def body(row, carry):
                rr = c * _RB + row
                wb = wclamp(m0 + rr)
                for t in range(8):
                    rows_v[bufrow0 + row, pl.ds(wb + 16 * t, 16)] = zeros16
                return carry
            lax.fori_loop(0, _RB, body, 0)

        handles = {}
        for c in range(_NCHUNK):
            bufrow0 = (c % 2) * _RB
            if c >= 2:
                handles[c - 2].wait()
                clear_chunk(c - 2, bufrow0)
            build_chunk(c, bufrow0)
            handles[c] = pltpu.async_copy(
                rows_v.at[pl.ds(bufrow0, _RB), :],
                att_hbm.at[pl.ds(base + c * _RB, _RB), :],
                sems[c % 2])
        handles[_NCHUNK - 2].wait()
        handles[_NCHUNK - 1].wait()

    return build


def kernel(x, Wq, bq, Wk, bk, Wv, bv, gamma):
    B, C, width, height = x.shape
    N = width * height
    xf = x.reshape(B, C, N)
    attn16, vals = _attn_call(xf, Wq, bq, Wk, bk, width, height)
    zeros = jnp.zeros((_RB, N), jnp.float32)
    att = _make_sc_builder(B, N)(vals.reshape(B * N, 128), zeros)
    out3 = _out_call(xf, attn16, Wv, bv, gamma, width, height)
    return out3.reshape(B, C, width, height), att.reshape(B, N, N)


# speedup vs baseline: 1.4632x; 1.0257x over previous
import functools

import jax
import jax.numpy as jnp
import numpy as np
from jax import lax
from jax.experimental import pallas as pl
from jax.experimental.pallas import tpu as pltpu
from jax.experimental.pallas import tpu_sc as plsc

_OFFS = tuple((dr, dc) for dr in (-1, 0, 1) for dc in (-1, 0, 1))


def _roll_lanes(a, shift):
    s = shift % a.shape[-1]
    if s == 0:
        return a
    return jnp.concatenate([a[:, s:], a[:, :s]], axis=1)


def _attn_body(x_ref, wq_ref, bq_ref, wk_ref, bk_ref, s_ref,
               attn_ref, vals_ref, *, width, height):
    n_pos = width * height
    xf = x_ref[0]
    q = jnp.dot(wq_ref[...], xf, preferred_element_type=jnp.float32) + bq_ref[...]
    k = jnp.dot(wk_ref[...], xf, preferred_element_type=jnp.float32) + bk_ref[...]

    n_iota = jax.lax.broadcasted_iota(jnp.int32, (1, n_pos), 1)
    r = n_iota // height
    c = n_iota % height

    energies = []
    for dr, dc in _OFFS:
        off = dr * height + dc
        kr = _roll_lanes(k, off)
        e = jnp.sum(q * kr, axis=0, keepdims=True)
        valid = ((r + dr >= 0) & (r + dr < width)
                 & (c + dc >= 0) & (c + dc < height))
        energies.append(jnp.where(valid, e, -1e30))
    energy = jnp.concatenate(energies, axis=0)
    emax = jnp.max(energy, axis=0, keepdims=True)
    p = jnp.exp(energy - emax)
    attn = p / jnp.sum(p, axis=0, keepdims=True)
    attn16 = jnp.concatenate(
        [attn, jnp.zeros((16 - attn.shape[0], n_pos), jnp.float32)], axis=0)
    attn_ref[0] = attn16

    attn_t = lax.dot_general(
        attn16, s_ref[...], (((0,), (0,)), ((), ())),
        preferred_element_type=jnp.float32)
    nrow = jax.lax.broadcasted_iota(jnp.int32, (n_pos, 128), 0)
    jcol = jax.lax.broadcasted_iota(jnp.int32, (n_pos, 128), 1)
    wb = jnp.minimum(jnp.maximum(((nrow - 33) // 16) * 16, 0), n_pos - 128)
    c0 = jcol + wb - nrow
    vals = jnp.zeros((n_pos, 128), jnp.float32)
    for i, (dr, dc) in enumerate(_OFFS):
        off = dr * height + dc
        vals = jnp.where(c0 == off, attn_t[:, i:i + 1], vals)
    vals_ref[0] = vals


def _out_body(x_ref, attn_ref, wv_ref, bv_ref, g_ref, out_ref,
              *, width, height):
    xf = x_ref[0]
    v = jnp.dot(wv_ref[...], xf, preferred_element_type=jnp.float32) + bv_ref[...]
    attn = attn_ref[0]
    v_dc = {dc: _roll_lanes(v, dc) for dc in (-1, 0, 1)}
    acc = jnp.zeros_like(v)
    for dr in (-1, 0, 1):
        w = jnp.zeros_like(v)
        for dc in (-1, 0, 1):
            i = (dr + 1) * 3 + (dc + 1)
            w = w + _roll_lanes(attn[i:i + 1, :], -dr * height) * v_dc[dc]
        acc = acc + _roll_lanes(w, dr * height)
    out_ref[0] = g_ref[0, 0] * acc + xf


def _attn_call(xf, Wq, bq, Wk, bk, width, height):
    B, C, N = xf.shape
    d = Wq.shape[0]
    body = functools.partial(_attn_body, width=width, height=height)
    return pl.pallas_call(
        body,
        grid=(B,),
        in_specs=[
            pl.BlockSpec((1, C, N), lambda b: (b, 0, 0)),
            pl.BlockSpec((d, C), lambda b: (0, 0)),
            pl.BlockSpec((d, 1), lambda b: (0, 0)),
            pl.BlockSpec((d, C), lambda b: (0, 0)),
            pl.BlockSpec((d, 1), lambda b: (0, 0)),
            pl.BlockSpec((16, 16), lambda b: (0, 0)),
        ],
        out_specs=[
            pl.BlockSpec((1, 16, N), lambda b: (b, 0, 0)),
            pl.BlockSpec((1, N, 128), lambda b: (b, 0, 0)),
        ],
        out_shape=[
            jax.ShapeDtypeStruct((B, 16, N), jnp.float32),
            jax.ShapeDtypeStruct((B, N, 128), jnp.float32),
        ],
    )(xf, Wq, bq.reshape(d, 1), Wk, bk.reshape(d, 1),
      jnp.eye(16, dtype=jnp.float32))


def _out_call(xf, attn16, Wv, bv, gamma, width, height):
    B, C, N = xf.shape
    body = functools.partial(_out_body, width=width, height=height)
    return pl.pallas_call(
        body,
        grid=(B,),
        in_specs=[
            pl.BlockSpec((1, C, N), lambda b: (b, 0, 0)),
            pl.BlockSpec((1, 16, N), lambda b: (b, 0, 0)),
            pl.BlockSpec((C, C), lambda b: (0, 0)),
            pl.BlockSpec((C, 1), lambda b: (0, 0)),
            pl.BlockSpec((1, 1), lambda b: (0, 0)),
        ],
        out_specs=pl.BlockSpec((1, C, N), lambda b: (b, 0, 0)),
        out_shape=jax.ShapeDtypeStruct((B, C, N), jnp.float32),
    )(xf, attn16, Wv, bv.reshape(C, 1), gamma.reshape(1, 1))


_RB = 32
_NCHUNK = 8


def _make_sc_builder(B, N):
    info = plsc.get_sparse_core_info()
    NC, NS = info.num_cores, info.num_subcores
    NW = NC * NS
    rows_w = (B * N) // NW
    mesh = plsc.VectorSubcoreMesh(core_axis_name="c", subcore_axis_name="s")

    @functools.partial(
        pl.kernel, mesh=mesh,
        out_type=jax.ShapeDtypeStruct((B * N, N), jnp.float32),
        scratch_types=[
            pltpu.VMEM((rows_w, 128), jnp.float32),
            pltpu.VMEM((2 * _RB, N), jnp.float32),
            pltpu.SemaphoreType.DMA,
            pltpu.SemaphoreType.DMA,
        ],
    )
    def build(vals_hbm, zeros_hbm, att_hbm, vals_v, rows_v, sem0, sem1):
        wid = lax.axis_index("s") * NC + lax.axis_index("c")
        base = wid * rows_w
        m0 = base % N

        pltpu.sync_copy(vals_hbm.at[pl.ds(base, rows_w), :], vals_v)
        pltpu.sync_copy(zeros_hbm, rows_v.at[pl.ds(0, _RB), :])
        pltpu.sync_copy(zeros_hbm, rows_v.at[pl.ds(_RB, _RB), :])

        zeros16 = jnp.zeros((16,), jnp.float32)
        sems = (sem0, sem1)

        def wclamp(m):
            wb = jnp.minimum(jnp.maximum(((m - 33) // 16) * 16, 0), N - 128)
            return pl.multiple_of(wb, 16)

        def build_chunk(c, bufrow0):
            def body(row, carry):
                rr = c * _RB + row
                wb = wclamp(m0 + rr)
                for t in range(8):
                    g = vals_v[rr, pl.ds(16 * t, 16)]
                    rows_v[bufrow0 + row, pl.ds(wb + 16 * t, 16)] = g
                return carry
            lax.fori_loop(0, _RB, body, 0)

        def clear_chunk(c, bufrow0):
            def body(row, carry):
                rr = c * _RB + row
                wb = wclamp(m0 + rr)
                for t in range(8):
                    rows_v[bufrow0 + row, pl.ds(wb + 16 * t, 16)] = zeros16
                return carry
            lax.fori_loop(0, _RB, body, 0)

        handles = {}
        for c in range(_NCHUNK):
            bufrow0 = (c % 2) * _RB
            if c >= 2:
                handles[c - 2].wait()
                clear_chunk(c - 2, bufrow0)
            build_chunk(c, bufrow0)
            handles[c] = pltpu.async_copy(
                rows_v.at[pl.ds(bufrow0, _RB), :],
                att_hbm.at[pl.ds(base + c * _RB, _RB), :],
                sems[c % 2])
        handles[_NCHUNK - 2].wait()
        handles[_NCHUNK - 1].wait()

    return build


def kernel(x, Wq, bq, Wk, bk, Wv, bv, gamma):
    B, C, width, height = x.shape
    N = width * height
    xf = x.reshape(B, C, N)
    attn16, vals = _attn_call(xf, Wq, bq, Wk, bk, width, height)
    zeros = jnp.zeros((_RB, N), jnp.float32)
    att = _make_sc_builder(B, N)(vals.reshape(B * N, 128), zeros)
    out3 = _out_call(xf, attn16, Wv, bv, gamma, width, height)
    return out3.reshape(B, C, width, height), att.reshape(B, N, N)


# speedup vs baseline: 1.4641x; 1.0006x over previous
import functools

import jax
import jax.numpy as jnp
from jax import lax
from jax.experimental import pallas as pl
from jax.experimental.pallas import tpu as pltpu
from jax.experimental.pallas import tpu_sc as plsc

_OFFS = tuple((dr, dc) for dr in (-1, 0, 1) for dc in (-1, 0, 1))


def _roll_lanes(a, shift):
    s = shift % a.shape[-1]
    if s == 0:
        return a
    return jnp.concatenate([a[:, s:], a[:, :s]], axis=1)


def _attn_body(x_ref, wq_ref, bq_ref, wk_ref, bk_ref, s_ref,
               attn_ref, vals_ref, *, width, height):
    n_pos = width * height
    xf = x_ref[0]
    q = jnp.dot(wq_ref[...], xf, preferred_element_type=jnp.float32) + bq_ref[...]
    k = jnp.dot(wk_ref[...], xf, preferred_element_type=jnp.float32) + bk_ref[...]

    n_iota = jax.lax.broadcasted_iota(jnp.int32, (1, n_pos), 1)
    r = n_iota // height
    c = n_iota % height

    energies = []
    for dr, dc in _OFFS:
        off = dr * height + dc
        kr = _roll_lanes(k, off)
        e = jnp.sum(q * kr, axis=0, keepdims=True)
        valid = ((r + dr >= 0) & (r + dr < width)
                 & (c + dc >= 0) & (c + dc < height))
        energies.append(jnp.where(valid, e, -1e30))
    energy = jnp.concatenate(energies, axis=0)
    emax = jnp.max(energy, axis=0, keepdims=True)
    p = jnp.exp(energy - emax)
    attn = p / jnp.sum(p, axis=0, keepdims=True)
    attn16 = jnp.concatenate(
        [attn, jnp.zeros((16 - attn.shape[0], n_pos), jnp.float32)], axis=0)
    attn_ref[0] = attn16

    attn_t = lax.dot_general(
        attn16, s_ref[...], (((0,), (0,)), ((), ())),
        preferred_element_type=jnp.float32)
    nrow = jax.lax.broadcasted_iota(jnp.int32, (n_pos, 128), 0)
    jcol = jax.lax.broadcasted_iota(jnp.int32, (n_pos, 128), 1)
    wb = jnp.minimum(jnp.maximum(((nrow - 33) // 16) * 16, 0), n_pos - 128)
    c0 = jcol + wb - nrow
    vals = jnp.zeros((n_pos, 128), jnp.float32)
    for i, (dr, dc) in enumerate(_OFFS):
        off = dr * height + dc
        vals = jnp.where(c0 == off, attn_t[:, i:i + 1], vals)
    vals_ref[0] = vals


def _out_body(x_ref, attn_ref, wv_ref, bv_ref, g_ref, out_ref,
              *, width, height):
    xf = x_ref[0]
    v = jnp.dot(wv_ref[...], xf, preferred_element_type=jnp.float32) + bv_ref[...]
    attn = attn_ref[0]
    v_dc = {dc: _roll_lanes(v, dc) for dc in (-1, 0, 1)}
    acc = jnp.zeros_like(v)
    for dr in (-1, 0, 1):
        w = jnp.zeros_like(v)
        for dc in (-1, 0, 1):
            i = (dr + 1) * 3 + (dc + 1)
            w = w + _roll_lanes(attn[i:i + 1, :], -dr * height) * v_dc[dc]
        acc = acc + _roll_lanes(w, dr * height)
    out_ref[0] = g_ref[0, 0] * acc + xf


def _attn_call(xf, Wq, bq, Wk, bk, width, height):
    B, C, N = xf.shape
    d = Wq.shape[0]
    body = functools.partial(_attn_body, width=width, height=height)
    return pl.pallas_call(
        body,
        grid=(B,),
        in_specs=[
            pl.BlockSpec((1, C, N), lambda b: (b, 0, 0)),
            pl.BlockSpec((d, C), lambda b: (0, 0)),
            pl.BlockSpec((d, 1), lambda b: (0, 0)),
            pl.BlockSpec((d, C), lambda b: (0, 0)),
            pl.BlockSpec((d, 1), lambda b: (0, 0)),
            pl.BlockSpec((16, 16), lambda b: (0, 0)),
        ],
        out_specs=[
            pl.BlockSpec((1, 16, N), lambda b: (b, 0, 0)),
            pl.BlockSpec((1, N, 128), lambda b: (b, 0, 0)),
        ],
        out_shape=[
            jax.ShapeDtypeStruct((B, 16, N), jnp.float32),
            jax.ShapeDtypeStruct((B, N, 128), jnp.float32),
        ],
    )(xf, Wq, bq.reshape(d, 1), Wk, bk.reshape(d, 1),
      jnp.eye(16, dtype=jnp.float32))


def _out_call(xf, attn16, Wv, bv, gamma, width, height):
    B, C, N = xf.shape
    body = functools.partial(_out_body, width=width, height=height)
    return pl.pallas_call(
        body,
        grid=(B,),
        in_specs=[
            pl.BlockSpec((1, C, N), lambda b: (b, 0, 0)),
            pl.BlockSpec((1, 16, N), lambda b: (b, 0, 0)),
            pl.BlockSpec((C, C), lambda b: (0, 0)),
            pl.BlockSpec((C, 1), lambda b: (0, 0)),
            pl.BlockSpec((1, 1), lambda b: (0, 0)),
        ],
        out_specs=pl.BlockSpec((1, C, N), lambda b: (b, 0, 0)),
        out_shape=jax.ShapeDtypeStruct((B, C, N), jnp.float32),
    )(xf, attn16, Wv, bv.reshape(C, 1), gamma.reshape(1, 1))


_RB = 32
_NCHUNK = 8


def _make_sc_builder(B, N):
    info = plsc.get_sparse_core_info()
    NC, NS = info.num_cores, info.num_subcores
    NW = NC * NS
    rows_w = (B * N) // NW
    mesh = plsc.VectorSubcoreMesh(core_axis_name="c", subcore_axis_name="s")

    @functools.partial(
        pl.kernel, mesh=mesh,
        out_type=jax.ShapeDtypeStruct((B * N, N), jnp.float32),
        scratch_types=[
            pltpu.VMEM((rows_w, 128), jnp.float32),
            pltpu.VMEM((2 * _RB, N), jnp.float32),
            pltpu.SemaphoreType.DMA,
            pltpu.SemaphoreType.DMA,
        ],
    )
    def build(vals_hbm, zeros_hbm, att_hbm, vals_v, rows_v, sem0, sem1):
        wid = lax.axis_index("s") * NC + lax.axis_index("c")
        base = wid * rows_w
        m0 = base % N

        pltpu.sync_copy(vals_hbm.at[pl.ds(base, rows_w), :], vals_v)
        pltpu.sync_copy(zeros_hbm, rows_v.at[pl.ds(0, _RB), :])
        pltpu.sync_copy(zeros_hbm, rows_v.at[pl.ds(_RB, _RB), :])

        zeros16 = jnp.zeros((16,), jnp.float32)
        sems = (sem0, sem1)

        def wclamp(m):
            wb = jnp.minimum(jnp.maximum(((m - 33) // 16) * 16, 0), N - 128)
            return pl.multiple_of(wb, 16)

        def build_chunk(c, bufrow0):
            def body(row, carry):
                rr = c * _RB + row
                wb = wclamp(m0 + rr)
                for t in range(8):
                    g = vals_v[rr, pl.ds(16 * t, 16)]
                    rows_v[bufrow0 + row, pl.ds(wb + 16 * t, 16)] = g
                return carry
            lax.fori_loop(0, _RB, body, 0)

        def clear_chunk(c, bufrow0):
            def body(row, carry):
                rr = c * _RB + row
                wb = wclamp(m0 + rr)
                for t in range(8):
                    rows_v[bufrow0 + row, pl.ds(wb + 16 * t, 16)] = zeros16
                return carry
            lax.fori_loop(0, _RB, body, 0)

        handles = {}
        for c in range(_NCHUNK):
            bufrow0 = (c % 2) * _RB
            if c >= 2:
                handles[c - 2].wait()
                clear_chunk(c - 2, bufrow0)
            build_chunk(c, bufrow0)
            handles[c] = pltpu.async_copy(
                rows_v.at[pl.ds(bufrow0, _RB), :],
                att_hbm.at[pl.ds(base + c * _RB, _RB), :],
                sems[c % 2])
        handles[_NCHUNK - 2].wait()
        handles[_NCHUNK - 1].wait()

    return build


def kernel(x, Wq, bq, Wk, bk, Wv, bv, gamma):
    B, C, width, height = x.shape
    N = width * height
    xf = x.reshape(B, C, N)
    attn16, vals = _attn_call(xf, Wq, bq, Wk, bk, width, height)
    zeros = jnp.zeros((_RB, N), jnp.float32)
    att = _make_sc_builder(B, N)(vals.reshape(B * N, 128), zeros)
    out3 = _out_call(xf, attn16, Wv, bv, gamma, width, height)
    return out3.reshape(B, C, width, height), att.reshape(B, N, N)


# speedup vs baseline: 1.5096x; 1.0311x over previous
import functools

import jax
import jax.numpy as jnp
from jax import lax
from jax.experimental import pallas as pl
from jax.experimental.pallas import tpu as pltpu
from jax.experimental.pallas import tpu_sc as plsc

_OFFS = tuple((dr, dc) for dr in (-1, 0, 1) for dc in (-1, 0, 1))


def _roll_lanes(a, shift):
    s = shift % a.shape[-1]
    if s == 0:
        return a
    return jnp.concatenate([a[:, s:], a[:, :s]], axis=1)


def _attn_body(x_ref, wq_ref, bq_ref, wk_ref, bk_ref, s_ref,
               attn_ref, vals_ref, *, width, height):
    n_pos = width * height
    xf = x_ref[0]
    q = jnp.dot(wq_ref[...], xf, preferred_element_type=jnp.float32) + bq_ref[...]
    k = jnp.dot(wk_ref[...], xf, preferred_element_type=jnp.float32) + bk_ref[...]

    n_iota = jax.lax.broadcasted_iota(jnp.int32, (1, n_pos), 1)
    r = n_iota // height
    c = n_iota % height

    energies = []
    for dr, dc in _OFFS:
        off = dr * height + dc
        kr = _roll_lanes(k, off)
        e = jnp.sum(q * kr, axis=0, keepdims=True)
        valid = ((r + dr >= 0) & (r + dr < width)
                 & (c + dc >= 0) & (c + dc < height))
        energies.append(jnp.where(valid, e, -1e30))
    energy = jnp.concatenate(energies, axis=0)
    emax = jnp.max(energy, axis=0, keepdims=True)
    p = jnp.exp(energy - emax)
    attn = p / jnp.sum(p, axis=0, keepdims=True)
    attn16 = jnp.concatenate(
        [attn, jnp.zeros((16 - attn.shape[0], n_pos), jnp.float32)], axis=0)
    attn_ref[0] = attn16

    attn_t = lax.dot_general(
        attn16, s_ref[...], (((0,), (0,)), ((), ())),
        preferred_element_type=jnp.float32)
    nrow = jax.lax.broadcasted_iota(jnp.int32, (n_pos, 128), 0)
    jcol = jax.lax.broadcasted_iota(jnp.int32, (n_pos, 128), 1)
    wb = jnp.minimum(jnp.maximum(((nrow - 33) // 16) * 16, 0), n_pos - 128)
    c0 = jcol + wb - nrow
    vals = jnp.zeros((n_pos, 128), jnp.float32)
    for i, (dr, dc) in enumerate(_OFFS):
        off = dr * height + dc
        vals = jnp.where(c0 == off, attn_t[:, i:i + 1], vals)
    vals_ref[0] = vals


def _out_body(x_ref, attn_ref, wv_ref, bv_ref, g_ref, out_ref,
              *, width, height):
    xf = x_ref[0]
    v = jnp.dot(wv_ref[...], xf, preferred_element_type=jnp.float32) + bv_ref[...]
    attn = attn_ref[0]
    v_dc = {dc: _roll_lanes(v, dc) for dc in (-1, 0, 1)}
    acc = jnp.zeros_like(v)
    for dr in (-1, 0, 1):
        w = jnp.zeros_like(v)
        for dc in (-1, 0, 1):
            i = (dr + 1) * 3 + (dc + 1)
            w = w + _roll_lanes(attn[i:i + 1, :], -dr * height) * v_dc[dc]
        acc = acc + _roll_lanes(w, dr * height)
    out_ref[0] = g_ref[0, 0] * acc + xf


def _attn_call(xf, Wq, bq, Wk, bk, width, height):
    B, C, N = xf.shape
    d = Wq.shape[0]
    body = functools.partial(_attn_body, width=width, height=height)
    return pl.pallas_call(
        body,
        grid=(B,),
        in_specs=[
            pl.BlockSpec((1, C, N), lambda b: (b, 0, 0)),
            pl.BlockSpec((d, C), lambda b: (0, 0)),
            pl.BlockSpec((d, 1), lambda b: (0, 0)),
            pl.BlockSpec((d, C), lambda b: (0, 0)),
            pl.BlockSpec((d, 1), lambda b: (0, 0)),
            pl.BlockSpec((16, 16), lambda b: (0, 0)),
        ],
        out_specs=[
            pl.BlockSpec((1, 16, N), lambda b: (b, 0, 0)),
            pl.BlockSpec((1, N, 128), lambda b: (b, 0, 0)),
        ],
        out_shape=[
            jax.ShapeDtypeStruct((B, 16, N), jnp.float32),
            jax.ShapeDtypeStruct((B, N, 128), jnp.float32),
        ],
    )(xf, Wq, bq.reshape(d, 1), Wk, bk.reshape(d, 1),
      jnp.eye(16, dtype=jnp.float32))


def _out_call(xf, attn16, Wv, bv, gamma, width, height):
    B, C, N = xf.shape
    body = functools.partial(_out_body, width=width, height=height)
    return pl.pallas_call(
        body,
        grid=(B,),
        in_specs=[
            pl.BlockSpec((1, C, N), lambda b: (b, 0, 0)),
            pl.BlockSpec((1, 16, N), lambda b: (b, 0, 0)),
            pl.BlockSpec((C, C), lambda b: (0, 0)),
            pl.BlockSpec((C, 1), lambda b: (0, 0)),
            pl.BlockSpec((1, 1), lambda b: (0, 0)),
        ],
        out_specs=pl.BlockSpec((1, C, N), lambda b: (b, 0, 0)),
        out_shape=jax.ShapeDtypeStruct((B, C, N), jnp.float32),
    )(xf, attn16, Wv, bv.reshape(C, 1), gamma.reshape(1, 1))


_RB = 32
_NCHUNK = 8


def _make_sc_builder(B, N):
    info = plsc.get_sparse_core_info()
    NC, NS = info.num_cores, info.num_subcores
    NW = NC * NS
    rows_w = (B * N) // NW
    mesh = plsc.VectorSubcoreMesh(core_axis_name="c", subcore_axis_name="s")

    @functools.partial(
        pl.kernel, mesh=mesh,
        out_type=jax.ShapeDtypeStruct((B * N, N), jnp.float32),
        scratch_types=[
            pltpu.VMEM((rows_w, 128), jnp.float32),
            pltpu.VMEM((2 * _RB, N), jnp.float32),
            pltpu.SemaphoreType.DMA,
            pltpu.SemaphoreType.DMA,
        ],
    )
    def build(vals_hbm, zeros_hbm, att_hbm, vals_v, rows_v, sem0, sem1):
        wid = lax.axis_index("s") * NC + lax.axis_index("c")
        base = wid * rows_w
        m0 = base % N

        zeros16 = jnp.zeros((16,), jnp.float32)

        zcopy = pltpu.async_copy(zeros_hbm, rows_v.at[pl.ds(0, _RB), :], sem0)
        pltpu.sync_copy(vals_hbm.at[pl.ds(base, rows_w), :], vals_v)

        def zbody(row, carry):
            for t in range(N // 16):
                rows_v[_RB + row, pl.ds(16 * t, 16)] = zeros16
            return carry
        lax.fori_loop(0, _RB, zbody, 0)
        zcopy.wait()
        sems = (sem0, sem1)

        def wclamp(m):
            wb = jnp.minimum(jnp.maximum(((m - 33) // 16) * 16, 0), N - 128)
            return pl.multiple_of(wb, 16)

        def build_chunk(c, bufrow0):
            def body(row, carry):
                rr = c * _RB + row
                wb = wclamp(m0 + rr)
                for t in range(8):
                    g = vals_v[rr, pl.ds(16 * t, 16)]
                    rows_v[bufrow0 + row, pl.ds(wb + 16 * t, 16)] = g
                return carry
            lax.fori_loop(0, _RB, body, 0)

        def clear_chunk(c, bufrow0):
            def body(row, carry):
                rr = c * _RB + row
                wb = wclamp(m0 + rr)
                for t in range(8):
                    rows_v[bufrow0 + row, pl.ds(wb + 16 * t, 16)] = zeros16
                return carry
            lax.fori_loop(0, _RB, body, 0)

        handles = {}
        for c in range(_NCHUNK):
            bufrow0 = (c % 2) * _RB
            if c >= 2:
                handles[c - 2].wait()
                clear_chunk(c - 2, bufrow0)
            build_chunk(c, bufrow0)
            handles[c] = pltpu.async_copy(
                rows_v.at[pl.ds(bufrow0, _RB), :],
                att_hbm.at[pl.ds(base + c * _RB, _RB), :],
                sems[c % 2])
        handles[_NCHUNK - 2].wait()
        handles[_NCHUNK - 1].wait()

    return build


def kernel(x, Wq, bq, Wk, bk, Wv, bv, gamma):
    B, C, width, height = x.shape
    N = width * height
    xf = x.reshape(B, C, N)
    attn16, vals = _attn_call(xf, Wq, bq, Wk, bk, width, height)
    zeros = jnp.zeros((_RB, N), jnp.float32)
    att = _make_sc_builder(B, N)(vals.reshape(B * N, 128), zeros)
    out3 = _out_call(xf, attn16, Wv, bv, gamma, width, height)
    return out3.reshape(B, C, width, height), att.reshape(B, N, N)
